# baseline (speedup 1.0000x reference)
import functools

import jax
import jax.numpy as jnp
from jax import lax
from jax.experimental import pallas as pl
from jax.experimental.pallas import tpu as pltpu
from jax.experimental.pallas import tpu_sc as plsc

_WINDOW = 512
_CHUNK = 200
_NWORKERS = 32


def _mesh():
    return plsc.VectorSubcoreMesh(core_axis_name="core",
                                  subcore_axis_name="subcore")


def _sc_untile(W):
    v, d = W.shape
    n_chunks = pl.cdiv(v, _CHUNK)
    per_worker = pl.cdiv(n_chunks, _NWORKERS)
    n_iters = pl.cdiv(per_worker, 2) * 2

    @jax.jit
    @functools.partial(
        pl.kernel,
        out_type=jax.ShapeDtypeStruct((v * d,), W.dtype),
        mesh=_mesh(),
        scratch_types=[
            pltpu.VMEM((2, _CHUNK, d), W.dtype),
            pltpu.VMEM((_CHUNK * d,), W.dtype),
            pltpu.SemaphoreType.DMA,
        ],
    )
    def untile_kernel(w_hbm, o_hbm, buf2d, buf1d, s_in):
        wid = lax.axis_index("subcore") * 2 + lax.axis_index("core")

        def chunk_id(k):
            return wid + k * _NWORKERS

        def start_in(k, slot):
            @pl.when(chunk_id(k) < n_chunks)
            def _():
                c = chunk_id(k)
                pltpu.async_copy(
                    w_hbm.at[pl.ds(c * _CHUNK, _CHUNK), :], buf2d.at[slot],
                    s_in)

        def finish(k, slot):
            @pl.when(chunk_id(k) < n_chunks)
            def _():
                c = chunk_id(k)
                pltpu.make_async_copy(
                    w_hbm.at[pl.ds(c * _CHUNK, _CHUNK), :], buf2d.at[slot],
                    s_in).wait()

                @pl.loop(0, _CHUNK, step=4)
                def _(r):
                    for rr in range(4):
                        for q in range(d // 16):
                            buf1d[pl.ds((r + rr) * d + q * 16, 16)] = (
                                buf2d[slot, r + rr, pl.ds(q * 16, 16)])

                pltpu.sync_copy(buf1d,
                                o_hbm.at[pl.ds(c * _CHUNK * d, _CHUNK * d)])

        start_in(0, 0)
        start_in(1, 1)

        @pl.loop(0, n_iters, step=2)
        def _(k):
            for dlt in range(2):
                finish(k + dlt, dlt)
                start_in(k + dlt + 2, dlt)

    return untile_kernel(W)


def _sc_gather(Wlin, idx_flat, d):
    n = idx_flat.shape[0]
    idx2 = idx_flat.reshape(1, n)

    @jax.jit
    @functools.partial(
        pl.kernel,
        out_type=jax.ShapeDtypeStruct((n, d), Wlin.dtype),
        mesh=_mesh(),
        compiler_params=pltpu.CompilerParams(use_tc_tiling_on_sc=False),
    )
    def gather_kernel(w_hbm, i_hbm, o_hbm):
        def body(i_vmem, o_vmem):
            pltpu.sync_copy(w_hbm.at[i_vmem.at[0]], o_vmem)

        pltpu.emit_pipeline(
            body,
            grid=(n // _WINDOW,),
            in_specs=[pl.BlockSpec((1, _WINDOW), index_map=lambda i: (0, i))],
            out_specs=[pl.BlockSpec((_WINDOW, d), index_map=lambda i: (i, 0))],
            core_axis_name=("core", "subcore"),
            dimension_semantics=(pltpu.PARALLEL,),
        )(i_hbm, o_hbm)

    return gather_kernel(Wlin, idx2)


def kernel(x, W):
    b, h = x.shape
    v, d = W.shape
    w_lin = _sc_untile(W).reshape(v, d)
    out = _sc_gather(w_lin, x.reshape(b * h).astype(jnp.int32), d)
    return out.reshape(b, h, d)


# speedup vs baseline: 1.2115x; 1.2115x over previous
import functools

import jax
import jax.numpy as jnp
from jax.experimental import pallas as pl
from jax.experimental.pallas import tpu as pltpu
from jax.experimental.pallas import tpu_sc as plsc

_WINDOW = 800


def _sc_gather(W, idx_flat):
    n = idx_flat.shape[0]
    d = W.shape[1]
    idx2 = idx_flat.reshape(1, n)
    mesh = plsc.VectorSubcoreMesh(core_axis_name="core",
                                  subcore_axis_name="subcore")

    @jax.jit
    @functools.partial(
        pl.kernel,
        out_type=jax.ShapeDtypeStruct((n, d), W.dtype),
        mesh=mesh,
        compiler_params=pltpu.CompilerParams(use_tc_tiling_on_sc=False),
    )
    def gather_kernel(w_hbm, i_hbm, o_hbm):
        def body(i_vmem, o_vmem):
            pltpu.sync_copy(w_hbm.at[i_vmem.at[0]], o_vmem)

        pltpu.emit_pipeline(
            body,
            grid=(n // _WINDOW,),
            in_specs=[pl.BlockSpec((1, _WINDOW), index_map=lambda i: (0, i))],
            out_specs=[pl.BlockSpec((_WINDOW, d), index_map=lambda i: (i, 0))],
            core_axis_name=("core", "subcore"),
            dimension_semantics=(pltpu.PARALLEL,),
        )(i_hbm, o_hbm)

    return gather_kernel(W, idx2)


def kernel(x, W):
    b, h = x.shape
    out = _sc_gather(W, x.reshape(b * h).astype(jnp.int32))
    return out.reshape(b, h, W.shape[1])
